# baseline (speedup 1.0000x reference)
import functools

import jax
import jax.numpy as jnp
from jax import lax
from jax.experimental import pallas as pl
from jax.experimental.pallas import tpu as pltpu
from jax.experimental.pallas import tpu_sc as plsc

_ROWS = 16384
_SEQ = 200
_D = 64
_NW = 32
_RPW = _ROWS // _NW
_R = 4
_C = _R * _SEQ
_C2 = _C // 2
_NCHUNK = _RPW // _R
_OUT_ROWS = _ROWS * _SEQ * _D // 128

_mesh = plsc.VectorSubcoreMesh(core_axis_name="c", subcore_axis_name="s")


@functools.partial(
    pl.kernel,
    mesh=_mesh,
    out_type=jax.ShapeDtypeStruct((_OUT_ROWS, 128), jnp.float32),
    scratch_types=[
        pltpu.VMEM((_C,), jnp.int32),
        pltpu.VMEM((_C,), jnp.int32),
        pltpu.VMEM((_C2,), jnp.int32),
        pltpu.VMEM((_C2,), jnp.int32),
        pltpu.VMEM((_C2,), jnp.int32),
        pltpu.VMEM((_C2,), jnp.int32),
        pltpu.VMEM((_C2, _D), jnp.float32),
        pltpu.VMEM((_C2, _D), jnp.float32),
        pltpu.VMEM((_C2, _D), jnp.float32),
        pltpu.VMEM((_C2, _D), jnp.float32),
        pltpu.SemaphoreType.DMA,
        pltpu.SemaphoreType.DMA,
        pltpu.SemaphoreType.DMA,
        pltpu.SemaphoreType.DMA,
        pltpu.SemaphoreType.DMA,
        pltpu.SemaphoreType.DMA,
    ],
    compiler_params=pltpu.CompilerParams(
        use_tc_tiling_on_sc=False, needs_layout_passes=False),
)
def _embed_sc(idx_hbm, table_hbm, out_hbm,
              idx_v0, idx_v1, ie0, ie1, io0, io1,
              re0, re1, ro0, ro1,
              si0, si1, sg0, sg1, so0, so1):
    wid = lax.axis_index("s") * 2 + lax.axis_index("c")
    base = wid * _RPW

    idx_v = (idx_v0, idx_v1)
    idx_e = (ie0, ie1)
    idx_o = (io0, io1)
    rows_e = (re0, re1)
    rows_o = (ro0, ro1)
    sem_i = (si0, si1)
    sem_g = (sg0, sg1)
    sem_o = (so0, so1)

    def idx_start(i, b):
        row0 = base + i * _R
        for k in range(_R):
            pltpu.make_async_copy(
                idx_hbm.at[row0 + k, :],
                idx_v[b].at[pl.ds(k * _SEQ, _SEQ)], sem_i[b]).start()

    def idx_wait(b):
        for k in range(_R):
            pltpu.make_async_copy(
                idx_hbm.at[0, :],
                idx_v[b].at[pl.ds(k * _SEQ, _SEQ)], sem_i[b]).wait()

    def deinterleave(b):
        lane = lax.iota(jnp.int32, 16) * 2
        for k in range(_C2 // 16):
            src = lane + (32 * k)
            idx_e[b][pl.ds(16 * k, 16)] = plsc.load_gather(idx_v[b], [src])
            idx_o[b][pl.ds(16 * k, 16)] = plsc.load_gather(idx_v[b], [src + 1])

    def gather_start(b):
        pltpu.make_async_copy(
            table_hbm.at[idx_e[b]], rows_e[b], sem_g[b]).start()
        pltpu.make_async_copy(
            table_hbm.at[idx_o[b]], rows_o[b], sem_g[b]).start()

    def gather_wait(b):
        pltpu.make_async_copy(
            table_hbm.at[idx_e[b]], rows_e[b], sem_g[b]).wait()
        pltpu.make_async_copy(
            table_hbm.at[idx_o[b]], rows_o[b], sem_g[b]).wait()

    def scatter_start(i, b):
        r0 = (base + i * _R) * (_SEQ * _D // 128)
        pltpu.make_async_copy(
            rows_e[b], out_hbm.at[pl.ds(r0, _C2), pl.ds(0, _D)],
            sem_o[b]).start()
        pltpu.make_async_copy(
            rows_o[b], out_hbm.at[pl.ds(r0, _C2), pl.ds(_D, _D)],
            sem_o[b]).start()

    def scatter_wait(b):
        pltpu.make_async_copy(
            rows_e[b], out_hbm.at[pl.ds(0, _C2), pl.ds(0, _D)],
            sem_o[b]).wait()
        pltpu.make_async_copy(
            rows_o[b], out_hbm.at[pl.ds(0, _C2), pl.ds(_D, _D)],
            sem_o[b]).wait()

    idx_start(0, 0)
    idx_wait(0)
    deinterleave(0)
    idx_start(1, 1)
    gather_start(0)

    @pl.loop(0, _NCHUNK, step=2)
    def _pair(i):
        for b in (0, 1):
            chunk = i + b
            nb = 1 - b

            @pl.when(chunk + 1 < _NCHUNK)
            def _():
                idx_wait(nb)
                deinterleave(nb)
                @pl.when(chunk >= 1)
                def _():
                    scatter_wait(nb)
                gather_start(nb)

            gather_wait(b)
            @pl.when(chunk + 2 < _NCHUNK)
            def _():
                idx_start(chunk + 2, b)
            scatter_start(chunk, b)

    scatter_wait(0)
    scatter_wait(1)


def kernel(input, weight):
    out = _embed_sc(jnp.asarray(input, jnp.int32), weight)
    return out.reshape(_ROWS, _SEQ, _D)
